# baseline (speedup 1.0000x reference)
import functools

import jax
import jax.numpy as jnp
from jax import lax
from jax.experimental import pallas as pl
from jax.experimental.pallas import tpu as pltpu
from jax.experimental.pallas import tpu_sc as plsc

N = 16384
D = 2048
NEXP = 16
BLK = 2048

NWORK = 32
RPW = N // NWORK
LANES = 16
NGRP = RPW // LANES


def _dense_block(h_ref, ww_ref, wn_ref, bs_ref, noise_ref, full_ref):
    h = h_ref[...]
    logits = jax.lax.dot_general(
        ww_ref[...], h, (((1,), (1,)), ((), ())),
        preferred_element_type=jnp.float32) + bs_ref[0:NEXP]
    nlin = jax.lax.dot_general(
        wn_ref[...], h, (((1,), (1,)), ((), ())),
        preferred_element_type=jnp.float32) + bs_ref[NEXP:2 * NEXP]
    noisy = logits + noise_ref[...] * jax.nn.softplus(nlin)

    m = jnp.max(noisy, axis=0, keepdims=True)
    e = jnp.exp(noisy - m)
    full_ref[...] = e / jnp.sum(e, axis=0, keepdims=True)


def _dense_full_t(h, W_w, b_w, W_n, b_n, noise_t):
    grid = (N // BLK,)

    return pl.pallas_call(
        _dense_block,
        grid=grid,
        in_specs=[
            pl.BlockSpec((BLK, D), lambda i: (i, 0)),
            pl.BlockSpec((NEXP, D), lambda i: (0, 0)),
            pl.BlockSpec((NEXP, D), lambda i: (0, 0)),
            pl.BlockSpec((2 * NEXP, 1), lambda i: (0, 0)),
            pl.BlockSpec((NEXP, BLK), lambda i: (0, i)),
        ],
        out_specs=pl.BlockSpec((NEXP, BLK), lambda i: (0, i)),
        out_shape=jax.ShapeDtypeStruct((NEXP, N), jnp.float32),
    )(h, W_w, W_n,
      jnp.concatenate([b_w, b_n]).reshape(2 * NEXP, 1), noise_t)


@functools.partial(
    pl.kernel,
    mesh=plsc.VectorSubcoreMesh(core_axis_name="c", subcore_axis_name="s"),
    out_type=[
        jax.ShapeDtypeStruct((NEXP, N), jnp.float32),
        jax.ShapeDtypeStruct((2, N), jnp.int32),
    ],
    scratch_types=[
        pltpu.VMEM((NEXP, RPW), jnp.float32),
        pltpu.VMEM((NEXP, RPW), jnp.float32),
        pltpu.VMEM((2, RPW), jnp.int32),
    ],
    compiler_params=pltpu.CompilerParams(
        needs_layout_passes=False,
        disable_bounds_checks=True,
        disable_semaphore_checks=True,
        skip_device_barrier=True,
    ),
)
def _sc_route(full_hbm, sparse_hbm, ix_hbm, fl_v, sp_v, ix_v):
    wid = lax.axis_index("s") * 2 + lax.axis_index("c")
    base = wid * RPW
    pltpu.sync_copy(full_hbm.at[:, pl.ds(base, RPW)], fl_v)

    lane = lax.broadcasted_iota(jnp.int32, (LANES,), 0)
    zeros_i = jnp.zeros((LANES,), jnp.int32)
    zeros_f = jnp.zeros((LANES,), jnp.float32)
    neg_inf = jnp.full((LANES,), -jnp.inf, jnp.float32)

    def group(g, _):
        cols = g * LANES + lane
        m1 = fl_v[0, pl.ds(g * LANES, LANES)]
        i1 = zeros_i
        m2 = neg_inf
        i2 = zeros_i
        sp_v[0, pl.ds(g * LANES, LANES)] = zeros_f
        for e in range(1, NEXP):
            v = fl_v[e, pl.ds(g * LANES, LANES)]
            sp_v[e, pl.ds(g * LANES, LANES)] = zeros_f
            gt1 = v > m1
            gt2 = v > m2
            i2 = jnp.where(gt1, i1, jnp.where(gt2, e, i2))
            m2 = jnp.where(gt1, m1, jnp.where(gt2, v, m2))
            i1 = jnp.where(gt1, e, i1)
            m1 = jnp.where(gt1, v, m1)
        s = m1 + m2
        plsc.store_scatter(sp_v, [i1, cols], m1 / s)
        plsc.store_scatter(sp_v, [i2, cols], m2 / s)
        ix_v[0, pl.ds(g * LANES, LANES)] = i1
        ix_v[1, pl.ds(g * LANES, LANES)] = i2
        return 0

    lax.fori_loop(0, NGRP, group, 0)

    pltpu.sync_copy(sp_v, sparse_hbm.at[:, pl.ds(base, RPW)])
    pltpu.sync_copy(ix_v, ix_hbm.at[:, pl.ds(base, RPW)])


@jax.jit
def _router(h, W_w, b_w, W_n, b_n, noise):
    full_t = _dense_full_t(h, W_w, b_w, W_n, b_n, noise.T)
    sparse_t, ix_t = _sc_route(full_t)
    return sparse_t.T, ix_t.T, full_t.T


def kernel(h, W_w, b_w, W_n, b_n, noise):
    return _router(h, W_w, b_w, W_n, b_n, noise)


# speedup vs baseline: 1.0021x; 1.0021x over previous
import functools

import jax
import jax.numpy as jnp
from jax import lax
from jax.experimental import pallas as pl
from jax.experimental.pallas import tpu as pltpu
from jax.experimental.pallas import tpu_sc as plsc

N = 16384
D = 2048
NEXP = 16
BLK = 2048

NWORK = 32
RPW = N // NWORK
LANES = 16
NGRP = RPW // LANES


def _dense_block(h_ref, ww_ref, wn_ref, bs_ref, noise_ref, full_ref):
    h = h_ref[...]
    logits = jax.lax.dot_general(
        ww_ref[...], h, (((1,), (1,)), ((), ())),
        preferred_element_type=jnp.float32) + bs_ref[0:NEXP, 0:1]
    nlin = jax.lax.dot_general(
        wn_ref[...], h, (((1,), (1,)), ((), ())),
        preferred_element_type=jnp.float32) + bs_ref[NEXP:2 * NEXP, 0:1]
    noisy = logits + noise_ref[...] * jax.nn.softplus(nlin)

    m = jnp.max(noisy, axis=0, keepdims=True)
    e = jnp.exp(noisy - m)
    full_ref[...] = e / jnp.sum(e, axis=0, keepdims=True)


def _dense_full_t(h, W_w, b_w, W_n, b_n, noise_t):
    grid = (N // BLK,)

    return pl.pallas_call(
        _dense_block,
        grid=grid,
        in_specs=[
            pl.BlockSpec((BLK, D), lambda i: (i, 0)),
            pl.BlockSpec((NEXP, D), lambda i: (0, 0)),
            pl.BlockSpec((NEXP, D), lambda i: (0, 0)),
            pl.BlockSpec((2 * NEXP, 128), lambda i: (0, 0)),
            pl.BlockSpec((NEXP, BLK), lambda i: (0, i)),
        ],
        out_specs=pl.BlockSpec((NEXP, BLK), lambda i: (0, i)),
        out_shape=jax.ShapeDtypeStruct((NEXP, N), jnp.float32),
    )(h, W_w, W_n,
      jnp.broadcast_to(jnp.concatenate([b_w, b_n])[:, None],
                       (2 * NEXP, 128)), noise_t)


@functools.partial(
    pl.kernel,
    mesh=plsc.VectorSubcoreMesh(core_axis_name="c", subcore_axis_name="s"),
    out_type=[
        jax.ShapeDtypeStruct((NEXP, N), jnp.float32),
        jax.ShapeDtypeStruct((2, N), jnp.int32),
    ],
    scratch_types=[
        pltpu.VMEM((NEXP, RPW), jnp.float32),
        pltpu.VMEM((NEXP, RPW), jnp.float32),
        pltpu.VMEM((2, RPW), jnp.int32),
    ],
    compiler_params=pltpu.CompilerParams(
        needs_layout_passes=False,
        disable_bounds_checks=True,
        disable_semaphore_checks=True,
        skip_device_barrier=True,
    ),
)
def _sc_route(full_hbm, sparse_hbm, ix_hbm, fl_v, sp_v, ix_v):
    wid = lax.axis_index("s") * 2 + lax.axis_index("c")
    base = wid * RPW
    pltpu.sync_copy(full_hbm.at[:, pl.ds(base, RPW)], fl_v)

    lane = lax.broadcasted_iota(jnp.int32, (LANES,), 0)
    zeros_i = jnp.zeros((LANES,), jnp.int32)
    zeros_f = jnp.zeros((LANES,), jnp.float32)
    neg_inf = jnp.full((LANES,), -jnp.inf, jnp.float32)

    def group(g, _):
        cols = g * LANES + lane
        m1 = fl_v[0, pl.ds(g * LANES, LANES)]
        i1 = zeros_i
        m2 = neg_inf
        i2 = zeros_i
        sp_v[0, pl.ds(g * LANES, LANES)] = zeros_f
        for e in range(1, NEXP):
            v = fl_v[e, pl.ds(g * LANES, LANES)]
            sp_v[e, pl.ds(g * LANES, LANES)] = zeros_f
            gt1 = v > m1
            gt2 = v > m2
            i2 = jnp.where(gt1, i1, jnp.where(gt2, e, i2))
            m2 = jnp.where(gt1, m1, jnp.where(gt2, v, m2))
            i1 = jnp.where(gt1, e, i1)
            m1 = jnp.where(gt1, v, m1)
        s = m1 + m2
        plsc.store_scatter(sp_v, [i1, cols], m1 / s)
        plsc.store_scatter(sp_v, [i2, cols], m2 / s)
        ix_v[0, pl.ds(g * LANES, LANES)] = i1
        ix_v[1, pl.ds(g * LANES, LANES)] = i2
        return 0

    lax.fori_loop(0, NGRP, group, 0)

    pltpu.sync_copy(sp_v, sparse_hbm.at[:, pl.ds(base, RPW)])
    pltpu.sync_copy(ix_v, ix_hbm.at[:, pl.ds(base, RPW)])


@jax.jit
def _router(h, W_w, b_w, W_n, b_n, noise):
    full_t = _dense_full_t(h, W_w, b_w, W_n, b_n, noise.T)
    sparse_t, ix_t = _sc_route(full_t)
    return sparse_t.T, ix_t.T, full_t.T


def kernel(h, W_w, b_w, W_n, b_n, noise):
    return _router(h, W_w, b_w, W_n, b_n, noise)
